# baseline (speedup 1.0000x reference)
import functools

import jax
import jax.numpy as jnp
from jax import lax
from jax.experimental import pallas as pl
from jax.experimental.pallas import tpu as pltpu
from jax.experimental.pallas import tpu_sc as plsc

B, L, D = 4096, 200, 32
NC, NS = 2, 16
NW = NC * NS
BW = B // NW
LANES = 16
NBC = BW // LANES
TD = D // 8
SLAB = D * BW
LSTRIDE = TD * NW * 8 * 128

_mesh = plsc.VectorSubcoreMesh(
    core_axis_name="c", subcore_axis_name="s", num_cores=NC, num_subcores=NS
)


@functools.partial(
    pl.kernel,
    out_type=jax.ShapeDtypeStruct(((L + 1) * LSTRIDE,), jnp.float32),
    mesh=_mesh,
    scratch_types=[
        pltpu.VMEM((BW * L,), jnp.int32),
        pltpu.VMEM((BW * L,), jnp.int32),
        pltpu.VMEM((BW * L,), jnp.int32),
        pltpu.VMEM((BW, D), jnp.float32),
        pltpu.VMEM((BW, D), jnp.float32),
        pltpu.VMEM((BW, D), jnp.float32),
        pltpu.VMEM((BW, D), jnp.float32),
        pltpu.VMEM((L * D,), jnp.float32),
        pltpu.VMEM((D,), jnp.float32),
        pltpu.VMEM((SLAB,), jnp.float32),
        pltpu.VMEM((SLAB,), jnp.float32),
        pltpu.SemaphoreType.DMA,
        pltpu.SemaphoreType.DMA,
        pltpu.SemaphoreType.DMA,
        pltpu.SemaphoreType.DMA,
    ],
    compiler_params=pltpu.CompilerParams(use_tc_tiling_on_sc=False,
                                         needs_layout_passes=False),
)
def _emb_kernel(bin_ids_hbm, subbin_ids_hbm, pos_hbm, bin_tab_hbm,
                subbin_tab_hbm, cls_hbm, out_hbm,
                ids_src, bin_t, sub_t, bin_a, sub_a, bin_b, sub_b, pos_v,
                cls_v, acc_a, acc_b, sem_a, sem_b, sem_oa, sem_ob):
    wid = lax.axis_index("s") * NC + lax.axis_index("c")
    base = wid * BW

    iota = lax.iota(jnp.int32, LANES)
    iota_l = iota * L
    rowvs = [iota + bc * LANES for bc in range(NBC)]

    pltpu.sync_copy(pos_hbm.at[pl.ds(0, L * D)], pos_v)
    pltpu.sync_copy(cls_hbm, cls_v)

    def transpose_ids(src_hbm, dst_t):
        pltpu.sync_copy(src_hbm.at[pl.ds(base * L, BW * L)], ids_src)

        def t_body(l, carry):
            for bc in range(NBC):
                g = plsc.load_gather(ids_src, [iota_l + (bc * LANES * L + l)])
                dst_t[pl.ds(l * BW + bc * LANES, LANES)] = g
            return carry

        lax.fori_loop(0, L, t_body, 0)

    transpose_ids(bin_ids_hbm, bin_t)
    transpose_ids(subbin_ids_hbm, sub_t)

    def cls_body(d, carry):
        splat = plsc.load_gather(cls_v, [jnp.broadcast_to(d, (LANES,))])
        off_d = (d // 8) * 1024 + (d % 8) * 128
        for bc in range(NBC):
            acc_a[pl.ds(off_d + bc * LANES, LANES)] = splat
        return carry

    lax.fori_loop(0, D, cls_body, 0)
    for td in range(TD):
        pltpu.sync_copy(
            acc_a.at[pl.ds(td * 1024, 1024)],
            out_hbm.at[pl.ds(td * NW * 1024 + wid * 1024, 1024)])

    def fire_gather(l, bin_v, sub_v, sem):
        pltpu.async_copy(bin_tab_hbm.at[bin_t.at[pl.ds(l * BW, BW)]],
                         bin_v, sem)
        pltpu.async_copy(subbin_tab_hbm.at[sub_t.at[pl.ds(l * BW, BW)]],
                         sub_v, sem)

    def drain_gather(bin_v, sub_v, sem):
        pltpu.make_async_copy(bin_tab_hbm.at[pl.ds(0, BW)], bin_v, sem).wait()
        pltpu.make_async_copy(subbin_tab_hbm.at[pl.ds(0, BW)], sub_v,
                              sem).wait()

    def drain_out(acc_v, sem):
        pltpu.make_async_copy(acc_v, out_hbm.at[pl.ds(0, SLAB)], sem).wait()

    def compute(bin_v, sub_v, acc_v, l):
        def d_body(d, carry):
            colv = jnp.broadcast_to(d, (LANES,))
            p = plsc.load_gather(pos_v, [colv + l * D])
            off_d = (d // 8) * 1024 + (d % 8) * 128
            for bc in range(NBC):
                v = (plsc.load_gather(bin_v, [rowvs[bc], colv])
                     + plsc.load_gather(sub_v, [rowvs[bc], colv]) + p)
                acc_v[pl.ds(off_d + bc * LANES, LANES)] = v
            return carry

        lax.fori_loop(0, D, d_body, 0)

    def fire_out(acc_v, l, sem):
        for td in range(TD):
            pltpu.async_copy(
                acc_v.at[pl.ds(td * 1024, 1024)],
                out_hbm.at[pl.ds((l + 1) * LSTRIDE + td * NW * 1024
                                 + wid * 1024, 1024)], sem)

    fire_gather(0, bin_a, sub_a, sem_a)

    def pair_body(g, carry):
        l0 = 2 * g
        fire_gather(l0 + 1, bin_b, sub_b, sem_b)
        drain_gather(bin_a, sub_a, sem_a)

        @pl.when(g > 0)
        def _():
            drain_out(acc_a, sem_oa)

        compute(bin_a, sub_a, acc_a, l0)
        fire_out(acc_a, l0, sem_oa)

        @pl.when(g < L // 2 - 1)
        def _():
            fire_gather(l0 + 2, bin_a, sub_a, sem_a)

        drain_gather(bin_b, sub_b, sem_b)

        @pl.when(g > 0)
        def _():
            drain_out(acc_b, sem_ob)

        compute(bin_b, sub_b, acc_b, l0 + 1)
        fire_out(acc_b, l0 + 1, sem_ob)
        return carry

    lax.fori_loop(0, L // 2, pair_body, 0)
    drain_out(acc_a, sem_oa)
    drain_out(acc_b, sem_ob)


def kernel(bin_ids, subbin_ids, pos_table, bin_table, subbin_table, cls_table):
    flat = _emb_kernel(bin_ids.astype(jnp.int32).reshape(-1),
                       subbin_ids.astype(jnp.int32).reshape(-1),
                       pos_table.reshape(-1), bin_table, subbin_table,
                       cls_table.reshape(-1))
    out5 = flat.reshape(L + 1, TD, NW, 8, 128)
    return out5.transpose(2, 4, 0, 1, 3).reshape(B, L + 1, D)


# speedup vs baseline: 1.0007x; 1.0007x over previous
import functools

import jax
import jax.numpy as jnp
from jax import lax
from jax.experimental import pallas as pl
from jax.experimental.pallas import tpu as pltpu
from jax.experimental.pallas import tpu_sc as plsc

B, L, D = 4096, 200, 32
NC, NS = 2, 16
NW = NC * NS
BW = B // NW
LANES = 16
NBC = BW // LANES
TD = D // 8
SLAB = D * BW
LSTRIDE = TD * NW * 8 * 128

_mesh = plsc.VectorSubcoreMesh(
    core_axis_name="c", subcore_axis_name="s", num_cores=NC, num_subcores=NS
)


@functools.partial(
    pl.kernel,
    out_type=jax.ShapeDtypeStruct(((L + 1) * LSTRIDE,), jnp.float32),
    mesh=_mesh,
    scratch_types=[
        pltpu.VMEM((BW * L,), jnp.int32),
        pltpu.VMEM((BW * L,), jnp.int32),
        pltpu.VMEM((BW * L,), jnp.int32),
        pltpu.VMEM((BW, D), jnp.float32),
        pltpu.VMEM((BW, D), jnp.float32),
        pltpu.VMEM((BW, D), jnp.float32),
        pltpu.VMEM((BW, D), jnp.float32),
        pltpu.VMEM((L * D,), jnp.float32),
        pltpu.VMEM((D,), jnp.float32),
        pltpu.VMEM((SLAB,), jnp.float32),
        pltpu.VMEM((SLAB,), jnp.float32),
        pltpu.SemaphoreType.DMA,
        pltpu.SemaphoreType.DMA,
        pltpu.SemaphoreType.DMA,
        pltpu.SemaphoreType.DMA,
    ],
    compiler_params=pltpu.CompilerParams(use_tc_tiling_on_sc=False,
                                         needs_layout_passes=False),
)
def _emb_kernel(bin_ids_hbm, subbin_ids_hbm, pos_hbm, bin_tab_hbm,
                subbin_tab_hbm, cls_hbm, out_hbm,
                ids_src, bin_t, sub_t, bin_a, sub_a, bin_b, sub_b, pos_v,
                cls_v, acc_a, acc_b, sem_a, sem_b, sem_oa, sem_ob):
    wid = lax.axis_index("s") * NC + lax.axis_index("c")
    base = wid * BW

    iota = lax.iota(jnp.int32, LANES)
    iota_l = iota * L
    rowvs = [iota + bc * LANES for bc in range(NBC)]

    pltpu.sync_copy(pos_hbm.at[pl.ds(0, L * D)], pos_v)
    pltpu.sync_copy(cls_hbm, cls_v)

    def transpose_ids(src_hbm, dst_t):
        pltpu.sync_copy(src_hbm.at[pl.ds(base * L, BW * L)], ids_src)

        def t_body(l, carry):
            for bc in range(NBC):
                g = plsc.load_gather(ids_src, [iota_l + (bc * LANES * L + l)])
                dst_t[pl.ds(l * BW + bc * LANES, LANES)] = g
            return carry

        lax.fori_loop(0, L, t_body, 0)

    with jax.named_scope("tpose_ids"):
        transpose_ids(bin_ids_hbm, bin_t)
        transpose_ids(subbin_ids_hbm, sub_t)

    def cls_body(d, carry):
        splat = plsc.load_gather(cls_v, [jnp.broadcast_to(d, (LANES,))])
        off_d = (d // 8) * 1024 + (d % 8) * 128
        for bc in range(NBC):
            acc_a[pl.ds(off_d + bc * LANES, LANES)] = splat
        return carry

    lax.fori_loop(0, D, cls_body, 0)
    for td in range(TD):
        pltpu.sync_copy(
            acc_a.at[pl.ds(td * 1024, 1024)],
            out_hbm.at[pl.ds(td * NW * 1024 + wid * 1024, 1024)])

    def fire_gather(l, bin_v, sub_v, sem):
        pltpu.async_copy(bin_tab_hbm.at[bin_t.at[pl.ds(l * BW, BW)]],
                         bin_v, sem)
        pltpu.async_copy(subbin_tab_hbm.at[sub_t.at[pl.ds(l * BW, BW)]],
                         sub_v, sem)

    def drain_gather(bin_v, sub_v, sem):
        with jax.named_scope("gwait"):
            pltpu.make_async_copy(bin_tab_hbm.at[pl.ds(0, BW)], bin_v,
                                  sem).wait()
            pltpu.make_async_copy(subbin_tab_hbm.at[pl.ds(0, BW)], sub_v,
                                  sem).wait()

    def drain_out(acc_v, sem):
        pltpu.make_async_copy(acc_v, out_hbm.at[pl.ds(0, SLAB)], sem).wait()

    def compute(bin_v, sub_v, acc_v, l):
        scope = jax.named_scope("comp")
        scope.__enter__()

        def d_body(d, carry):
            colv = jnp.broadcast_to(d, (LANES,))
            p = plsc.load_gather(pos_v, [colv + l * D])
            off_d = (d // 8) * 1024 + (d % 8) * 128
            for bc in range(NBC):
                v = (plsc.load_gather(bin_v, [rowvs[bc], colv])
                     + plsc.load_gather(sub_v, [rowvs[bc], colv]) + p)
                acc_v[pl.ds(off_d + bc * LANES, LANES)] = v
            return carry

        lax.fori_loop(0, D, d_body, 0)
        scope.__exit__(None, None, None)

    def fire_out(acc_v, l, sem):
        for td in range(TD):
            pltpu.async_copy(
                acc_v.at[pl.ds(td * 1024, 1024)],
                out_hbm.at[pl.ds((l + 1) * LSTRIDE + td * NW * 1024
                                 + wid * 1024, 1024)], sem)

    fire_gather(0, bin_a, sub_a, sem_a)

    def pair_body(g, carry):
        l0 = 2 * g
        fire_gather(l0 + 1, bin_b, sub_b, sem_b)
        drain_gather(bin_a, sub_a, sem_a)

        @pl.when(g > 0)
        def _():
            drain_out(acc_a, sem_oa)

        compute(bin_a, sub_a, acc_a, l0)
        fire_out(acc_a, l0, sem_oa)

        @pl.when(g < L // 2 - 1)
        def _():
            fire_gather(l0 + 2, bin_a, sub_a, sem_a)

        drain_gather(bin_b, sub_b, sem_b)

        @pl.when(g > 0)
        def _():
            drain_out(acc_b, sem_ob)

        compute(bin_b, sub_b, acc_b, l0 + 1)
        fire_out(acc_b, l0 + 1, sem_ob)
        return carry

    lax.fori_loop(0, L // 2, pair_body, 0)
    drain_out(acc_a, sem_oa)
    drain_out(acc_b, sem_ob)


def kernel(bin_ids, subbin_ids, pos_table, bin_table, subbin_table, cls_table):
    flat = _emb_kernel(bin_ids.astype(jnp.int32).reshape(-1),
                       subbin_ids.astype(jnp.int32).reshape(-1),
                       pos_table.reshape(-1), bin_table, subbin_table,
                       cls_table.reshape(-1))
    out5 = flat.reshape(L + 1, TD, NW, 8, 128)
    return out5.transpose(2, 4, 0, 1, 3).reshape(B, L + 1, D)


# speedup vs baseline: 1.9499x; 1.9486x over previous
import functools

import jax
import jax.numpy as jnp
from jax import lax
from jax.experimental import pallas as pl
from jax.experimental.pallas import tpu as pltpu
from jax.experimental.pallas import tpu_sc as plsc

B, L, D = 4096, 200, 32
NC, NS = 2, 16
NW = NC * NS
BW = B // NW
LANES = 16
NBC = BW // LANES
TD = D // 8
SLAB = D * BW
LSTRIDE = TD * NW * 8 * 128
PITCH = D + 1

_mesh = plsc.VectorSubcoreMesh(
    core_axis_name="c", subcore_axis_name="s", num_cores=NC, num_subcores=NS
)


@functools.partial(
    pl.kernel,
    out_type=jax.ShapeDtypeStruct(((L + 1) * LSTRIDE,), jnp.float32),
    mesh=_mesh,
    scratch_types=[
        pltpu.VMEM((BW * L,), jnp.int32),
        pltpu.VMEM((BW * L,), jnp.int32),
        pltpu.VMEM((BW,), jnp.int32),
        pltpu.VMEM((BW,), jnp.int32),
        pltpu.VMEM((BW,), jnp.int32),
        pltpu.VMEM((BW,), jnp.int32),
        pltpu.VMEM((BW, D), jnp.float32),
        pltpu.VMEM((BW, D), jnp.float32),
        pltpu.VMEM((BW, D), jnp.float32),
        pltpu.VMEM((BW, D), jnp.float32),
        pltpu.VMEM((BW * PITCH,), jnp.float32),
        pltpu.VMEM((L * D,), jnp.float32),
        pltpu.VMEM((D,), jnp.float32),
        pltpu.VMEM((SLAB,), jnp.float32),
        pltpu.VMEM((SLAB,), jnp.float32),
        pltpu.SemaphoreType.DMA,
        pltpu.SemaphoreType.DMA,
        pltpu.SemaphoreType.DMA,
        pltpu.SemaphoreType.DMA,
    ],
    compiler_params=pltpu.CompilerParams(use_tc_tiling_on_sc=False,
                                         needs_layout_passes=False),
)
def _emb_kernel(bin_ids_hbm, subbin_ids_hbm, pos_hbm, bin_tab_hbm,
                subbin_tab_hbm, cls_hbm, out_hbm,
                bin_src, sub_src, ixb_a, ixs_a, ixb_b, ixs_b,
                bin_a, sub_a, bin_b, sub_b, pad_v, pos_v, cls_v,
                acc_a, acc_b, sem_a, sem_b, sem_oa, sem_ob):
    wid = lax.axis_index("s") * NC + lax.axis_index("c")
    base = wid * BW

    iota = lax.iota(jnp.int32, LANES)
    iota_l = iota * L
    iota_p = iota * PITCH

    pltpu.sync_copy(bin_ids_hbm.at[pl.ds(base * L, BW * L)], bin_src)
    pltpu.sync_copy(subbin_ids_hbm.at[pl.ds(base * L, BW * L)], sub_src)
    pltpu.sync_copy(pos_hbm.at[pl.ds(0, L * D)], pos_v)
    pltpu.sync_copy(cls_hbm, cls_v)

    def cls_body(d, carry):
        splat = plsc.load_gather(cls_v, [jnp.broadcast_to(d, (LANES,))])
        off_d = (d // 8) * 1024 + (d % 8) * 128
        for bc in range(NBC):
            acc_a[pl.ds(off_d + bc * LANES, LANES)] = splat
        return carry

    lax.fori_loop(0, D, cls_body, 0)
    for td in range(TD):
        pltpu.sync_copy(
            acc_a.at[pl.ds(td * 1024, 1024)],
            out_hbm.at[pl.ds(td * NW * 1024 + wid * 1024, 1024)])

    def build_idx(l, ixb, ixs):
        for bc in range(NBC):
            lanes = iota_l + (bc * LANES * L + l)
            ixb[pl.ds(bc * LANES, LANES)] = plsc.load_gather(bin_src, [lanes])
            ixs[pl.ds(bc * LANES, LANES)] = plsc.load_gather(sub_src, [lanes])

    def fire_gather(bin_v, sub_v, ixb, ixs, sem):
        pltpu.async_copy(bin_tab_hbm.at[ixb], bin_v, sem)
        pltpu.async_copy(subbin_tab_hbm.at[ixs], sub_v, sem)

    def drain_gather(bin_v, sub_v, sem):
        with jax.named_scope("gwait"):
            pltpu.make_async_copy(bin_tab_hbm.at[pl.ds(0, BW)], bin_v,
                                  sem).wait()
            pltpu.make_async_copy(subbin_tab_hbm.at[pl.ds(0, BW)], sub_v,
                                  sem).wait()

    def drain_out(acc_v, sem):
        with jax.named_scope("owait"):
            pltpu.make_async_copy(acc_v, out_hbm.at[pl.ds(0, SLAB)],
                                  sem).wait()

    def compute(bin_v, sub_v, acc_v, l):
        scope = jax.named_scope("sum")
        scope.__enter__()
        p0 = pos_v[pl.ds(l * D, LANES)]
        p1 = pos_v[pl.ds(l * D + LANES, LANES)]

        def b_body(b, carry):
            pad_v[pl.ds(b * PITCH, LANES)] = (
                bin_v[b, pl.ds(0, LANES)] + sub_v[b, pl.ds(0, LANES)] + p0)
            pad_v[pl.ds(b * PITCH + LANES, LANES)] = (
                bin_v[b, pl.ds(LANES, LANES)]
                + sub_v[b, pl.ds(LANES, LANES)] + p1)
            return carry

        lax.fori_loop(0, BW, b_body, 0)
        scope.__exit__(None, None, None)

        scope = jax.named_scope("tr")
        scope.__enter__()

        def d_body(d, carry):
            off_d = (d // 8) * 1024 + (d % 8) * 128
            for bc in range(NBC):
                v = plsc.load_gather(
                    pad_v, [iota_p + (bc * (LANES * PITCH) + d)])
                acc_v[pl.ds(off_d + bc * LANES, LANES)] = v
            return carry

        lax.fori_loop(0, D, d_body, 0)
        scope.__exit__(None, None, None)

    def fire_out(acc_v, l, sem):
        for td in range(TD):
            pltpu.async_copy(
                acc_v.at[pl.ds(td * 1024, 1024)],
                out_hbm.at[pl.ds((l + 1) * LSTRIDE + td * NW * 1024
                                 + wid * 1024, 1024)], sem)

    build_idx(0, ixb_a, ixs_a)
    fire_gather(bin_a, sub_a, ixb_a, ixs_a, sem_a)

    def pair_body(g, carry):
        l0 = 2 * g
        build_idx(l0 + 1, ixb_b, ixs_b)
        fire_gather(bin_b, sub_b, ixb_b, ixs_b, sem_b)
        drain_gather(bin_a, sub_a, sem_a)

        @pl.when(g > 0)
        def _():
            drain_out(acc_a, sem_oa)

        compute(bin_a, sub_a, acc_a, l0)
        fire_out(acc_a, l0, sem_oa)

        @pl.when(g < L // 2 - 1)
        def _():
            build_idx(l0 + 2, ixb_a, ixs_a)
            fire_gather(bin_a, sub_a, ixb_a, ixs_a, sem_a)

        drain_gather(bin_b, sub_b, sem_b)

        @pl.when(g > 0)
        def _():
            drain_out(acc_b, sem_ob)

        compute(bin_b, sub_b, acc_b, l0 + 1)
        fire_out(acc_b, l0 + 1, sem_ob)
        return carry

    lax.fori_loop(0, L // 2, pair_body, 0)
    drain_out(acc_a, sem_oa)
    drain_out(acc_b, sem_ob)


def kernel(bin_ids, subbin_ids, pos_table, bin_table, subbin_table, cls_table):
    flat = _emb_kernel(bin_ids.astype(jnp.int32).reshape(-1),
                       subbin_ids.astype(jnp.int32).reshape(-1),
                       pos_table.reshape(-1), bin_table, subbin_table,
                       cls_table.reshape(-1))
    out5 = flat.reshape(L + 1, TD, NW, 8, 128)
    return out5.transpose(2, 4, 0, 1, 3).reshape(B, L + 1, D)


# speedup vs baseline: 4.5135x; 2.3147x over previous
import functools

import jax
import jax.numpy as jnp
from jax import lax
from jax.experimental import pallas as pl
from jax.experimental.pallas import tpu as pltpu
from jax.experimental.pallas import tpu_sc as plsc

B, L, D = 4096, 200, 32
NC, NS = 2, 16
NW = NC * NS
BW = B // NW
LANES = 16
NBC = BW // LANES
TD = D // 8
SLAB = D * BW
LSTRIDE = TD * NW * 8 * 128
PITCH = D + 1

_mesh = plsc.VectorSubcoreMesh(
    core_axis_name="c", subcore_axis_name="s", num_cores=NC, num_subcores=NS
)


@functools.partial(
    pl.kernel,
    out_type=jax.ShapeDtypeStruct(((L + 1) * LSTRIDE,), jnp.float32),
    mesh=_mesh,
    scratch_types=[
        pltpu.VMEM((BW * L,), jnp.int32),
        pltpu.VMEM((BW * L,), jnp.int32),
        pltpu.VMEM((BW,), jnp.int32),
        pltpu.VMEM((BW,), jnp.int32),
        pltpu.VMEM((BW,), jnp.int32),
        pltpu.VMEM((BW,), jnp.int32),
        pltpu.VMEM((BW, D), jnp.float32),
        pltpu.VMEM((BW, D), jnp.float32),
        pltpu.VMEM((BW, D), jnp.float32),
        pltpu.VMEM((BW, D), jnp.float32),
        pltpu.VMEM((BW * PITCH,), jnp.float32),
        pltpu.VMEM((L * D,), jnp.float32),
        pltpu.VMEM((D,), jnp.float32),
        pltpu.VMEM((SLAB,), jnp.float32),
        pltpu.VMEM((SLAB,), jnp.float32),
        pltpu.SemaphoreType.DMA,
        pltpu.SemaphoreType.DMA,
        pltpu.SemaphoreType.DMA,
        pltpu.SemaphoreType.DMA,
    ],
    compiler_params=pltpu.CompilerParams(use_tc_tiling_on_sc=False,
                                         needs_layout_passes=False),
)
def _emb_kernel(bin_ids_hbm, subbin_ids_hbm, pos_hbm, bin_tab_hbm,
                subbin_tab_hbm, cls_hbm, out_hbm,
                bin_src, sub_src, ixb_a, ixs_a, ixb_b, ixs_b,
                bin_a, sub_a, bin_b, sub_b, pad_v, pos_v, cls_v,
                acc_a, acc_b, sem_a, sem_b, sem_oa, sem_ob):
    wid = lax.axis_index("s") * NC + lax.axis_index("c")
    base = wid * BW

    iota = lax.iota(jnp.int32, LANES)
    iota_l = iota * L
    iota_p = iota * PITCH

    pltpu.sync_copy(bin_ids_hbm.at[pl.ds(base * L, BW * L)], bin_src)
    pltpu.sync_copy(subbin_ids_hbm.at[pl.ds(base * L, BW * L)], sub_src)
    pltpu.sync_copy(pos_hbm.at[pl.ds(0, L * D)], pos_v)
    pltpu.sync_copy(cls_hbm, cls_v)

    def cls_body(d, carry):
        splat = plsc.load_gather(cls_v, [jnp.broadcast_to(d, (LANES,))])
        off_d = (d // 8) * 1024 + (d % 8) * 128
        for bc in range(NBC):
            acc_a[pl.ds(off_d + bc * LANES, LANES)] = splat
        return carry

    lax.fori_loop(0, D, cls_body, 0)
    for td in range(TD):
        pltpu.sync_copy(
            acc_a.at[pl.ds(td * 1024, 1024)],
            out_hbm.at[pl.ds(td * NW * 1024 + wid * 1024, 1024)])

    def build_idx(l, ixb, ixs):
        for bc in range(NBC):
            lanes = iota_l + (bc * LANES * L + l)
            ixb[pl.ds(bc * LANES, LANES)] = plsc.load_gather(bin_src, [lanes])
            ixs[pl.ds(bc * LANES, LANES)] = plsc.load_gather(sub_src, [lanes])

    def fire_gather(bin_v, sub_v, ixb, ixs, sem):
        pltpu.async_copy(bin_tab_hbm.at[ixb], bin_v, sem)
        pltpu.async_copy(subbin_tab_hbm.at[ixs], sub_v, sem)

    def drain_gather(bin_v, sub_v, sem):
        with jax.named_scope("gwait"):
            pltpu.make_async_copy(bin_tab_hbm.at[pl.ds(0, BW)], bin_v,
                                  sem).wait()
            pltpu.make_async_copy(subbin_tab_hbm.at[pl.ds(0, BW)], sub_v,
                                  sem).wait()

    def drain_out(acc_v, sem):
        with jax.named_scope("owait"):
            pltpu.make_async_copy(acc_v, out_hbm.at[pl.ds(0, SLAB)],
                                  sem).wait()

    def compute(bin_v, sub_v, acc_v, l):
        scope = jax.named_scope("sum")
        scope.__enter__()
        p0 = pos_v[pl.ds(l * D, LANES)]
        p1 = pos_v[pl.ds(l * D + LANES, LANES)]

        @plsc.parallel_loop(0, BW, 1, unroll=8)
        def b_body(b):
            pad_v[pl.ds(b * PITCH, LANES)] = (
                bin_v[b, pl.ds(0, LANES)] + sub_v[b, pl.ds(0, LANES)] + p0)
            pad_v[pl.ds(b * PITCH + LANES, LANES)] = (
                bin_v[b, pl.ds(LANES, LANES)]
                + sub_v[b, pl.ds(LANES, LANES)] + p1)

        scope.__exit__(None, None, None)

        scope = jax.named_scope("tr")
        scope.__enter__()

        @plsc.parallel_loop(0, D, 1, unroll=4)
        def d_body(d):
            off_d = (d // 8) * 1024 + (d % 8) * 128
            for bc in range(NBC):
                v = plsc.load_gather(
                    pad_v, [iota_p + (bc * (LANES * PITCH) + d)])
                acc_v[pl.ds(off_d + bc * LANES, LANES)] = v

        scope.__exit__(None, None, None)

    def fire_out(acc_v, l, sem):
        for td in range(TD):
            pltpu.async_copy(
                acc_v.at[pl.ds(td * 1024, 1024)],
                out_hbm.at[pl.ds((l + 1) * LSTRIDE + td * NW * 1024
                                 + wid * 1024, 1024)], sem)

    build_idx(0, ixb_a, ixs_a)
    fire_gather(bin_a, sub_a, ixb_a, ixs_a, sem_a)

    def pair_body(g, carry):
        l0 = 2 * g
        build_idx(l0 + 1, ixb_b, ixs_b)
        fire_gather(bin_b, sub_b, ixb_b, ixs_b, sem_b)
        drain_gather(bin_a, sub_a, sem_a)

        @pl.when(g > 0)
        def _():
            drain_out(acc_a, sem_oa)

        compute(bin_a, sub_a, acc_a, l0)
        fire_out(acc_a, l0, sem_oa)

        @pl.when(g < L // 2 - 1)
        def _():
            build_idx(l0 + 2, ixb_a, ixs_a)
            fire_gather(bin_a, sub_a, ixb_a, ixs_a, sem_a)

        drain_gather(bin_b, sub_b, sem_b)

        @pl.when(g > 0)
        def _():
            drain_out(acc_b, sem_ob)

        compute(bin_b, sub_b, acc_b, l0 + 1)
        fire_out(acc_b, l0 + 1, sem_ob)
        return carry

    lax.fori_loop(0, L // 2, pair_body, 0)
    drain_out(acc_a, sem_oa)
    drain_out(acc_b, sem_ob)


def kernel(bin_ids, subbin_ids, pos_table, bin_table, subbin_table, cls_table):
    flat = _emb_kernel(bin_ids.astype(jnp.int32).reshape(-1),
                       subbin_ids.astype(jnp.int32).reshape(-1),
                       pos_table.reshape(-1), bin_table, subbin_table,
                       cls_table.reshape(-1))
    out5 = flat.reshape(L + 1, TD, NW, 8, 128)
    return out5.transpose(2, 4, 0, 1, 3).reshape(B, L + 1, D)


# speedup vs baseline: 5.1039x; 1.1308x over previous
import functools

import jax
import jax.numpy as jnp
from jax import lax
from jax.experimental import pallas as pl
from jax.experimental.pallas import tpu as pltpu
from jax.experimental.pallas import tpu_sc as plsc

B, L, D = 4096, 200, 32
NC, NS = 2, 16
NW = NC * NS
BW = B // NW
LANES = 16
NBC = BW // LANES
TD = D // 8
SLAB = D * BW
LSTRIDE = TD * NW * 8 * 128
PITCH = D + 1
LT = L // 8

_mesh = plsc.VectorSubcoreMesh(
    core_axis_name="c", subcore_axis_name="s", num_cores=NC, num_subcores=NS
)


@functools.partial(
    pl.kernel,
    out_type=jax.ShapeDtypeStruct(((L + 1) * LSTRIDE,), jnp.float32),
    mesh=_mesh,
    scratch_types=[
        pltpu.VMEM((BW * L,), jnp.int32),
        pltpu.VMEM((BW * L,), jnp.int32),
        pltpu.VMEM((BW, D), jnp.float32),
        pltpu.VMEM((BW, D), jnp.float32),
        pltpu.VMEM((BW, D), jnp.float32),
        pltpu.VMEM((BW, D), jnp.float32),
        pltpu.VMEM((BW * PITCH,), jnp.float32),
        pltpu.VMEM((L * D,), jnp.float32),
        pltpu.VMEM((D,), jnp.float32),
        pltpu.VMEM((SLAB,), jnp.float32),
        pltpu.VMEM((SLAB,), jnp.float32),
        pltpu.SemaphoreType.DMA,
        pltpu.SemaphoreType.DMA,
        pltpu.SemaphoreType.DMA,
        pltpu.SemaphoreType.DMA,
        pltpu.SemaphoreType.DMA,
    ],
    compiler_params=pltpu.CompilerParams(use_tc_tiling_on_sc=False,
                                         needs_layout_passes=False),
)
def _emb_kernel(bin_ids_hbm, subbin_ids_hbm, pos_hbm, bin_tab_hbm,
                subbin_tab_hbm, cls_hbm, out_hbm,
                bin_t, sub_t, bin_a, sub_a, bin_b, sub_b, pad_v, pos_v,
                cls_v, acc_a, acc_b, sem_i, sem_a, sem_b, sem_oa, sem_ob):
    wid = lax.axis_index("s") * NC + lax.axis_index("c")

    iota = lax.iota(jnp.int32, LANES)
    iota_p = iota * PITCH

    for lt in range(LT):
        pltpu.async_copy(
            bin_ids_hbm.at[pl.ds(lt * (NW * 8 * 128) + wid * 1024, 1024)],
            bin_t.at[pl.ds(lt * 1024, 1024)], sem_i)
        pltpu.async_copy(
            subbin_ids_hbm.at[pl.ds(lt * (NW * 8 * 128) + wid * 1024, 1024)],
            sub_t.at[pl.ds(lt * 1024, 1024)], sem_i)
    pltpu.sync_copy(pos_hbm.at[pl.ds(0, L * D)], pos_v)
    pltpu.sync_copy(cls_hbm, cls_v)
    pltpu.make_async_copy(bin_ids_hbm.at[pl.ds(0, BW * L)], bin_t,
                          sem_i).wait()
    pltpu.make_async_copy(subbin_ids_hbm.at[pl.ds(0, BW * L)], sub_t,
                          sem_i).wait()

    def cls_body(d, carry):
        splat = plsc.load_gather(cls_v, [jnp.broadcast_to(d, (LANES,))])
        off_d = (d // 8) * 1024 + (d % 8) * 128
        for bc in range(NBC):
            acc_a[pl.ds(off_d + bc * LANES, LANES)] = splat
        return carry

    lax.fori_loop(0, D, cls_body, 0)
    for td in range(TD):
        pltpu.sync_copy(
            acc_a.at[pl.ds(td * 1024, 1024)],
            out_hbm.at[pl.ds(td * NW * 1024 + wid * 1024, 1024)])

    def fire_gather(l, bin_v, sub_v, sem):
        pltpu.async_copy(bin_tab_hbm.at[bin_t.at[pl.ds(l * BW, BW)]],
                         bin_v, sem)
        pltpu.async_copy(subbin_tab_hbm.at[sub_t.at[pl.ds(l * BW, BW)]],
                         sub_v, sem)

    def drain_gather(bin_v, sub_v, sem):
        pltpu.make_async_copy(bin_tab_hbm.at[pl.ds(0, BW)], bin_v, sem).wait()
        pltpu.make_async_copy(subbin_tab_hbm.at[pl.ds(0, BW)], sub_v,
                              sem).wait()

    def drain_out(acc_v, sem):
        pltpu.make_async_copy(acc_v, out_hbm.at[pl.ds(0, SLAB)], sem).wait()

    def stage_sum(bin_v, sub_v, l):
        p0 = pos_v[pl.ds(l * D, LANES)]
        p1 = pos_v[pl.ds(l * D + LANES, LANES)]

        @plsc.parallel_loop(0, BW, 1, unroll=8)
        def b_body(b):
            pad_v[pl.ds(b * PITCH, LANES)] = (
                bin_v[b, pl.ds(0, LANES)] + sub_v[b, pl.ds(0, LANES)] + p0)
            pad_v[pl.ds(b * PITCH + LANES, LANES)] = (
                bin_v[b, pl.ds(LANES, LANES)]
                + sub_v[b, pl.ds(LANES, LANES)] + p1)

    def stage_tiles(acc_v):
        @plsc.parallel_loop(0, D, 1, unroll=4)
        def d_body(d):
            off_d = (d // 8) * 1024 + (d % 8) * 128
            for bc in range(NBC):
                v = plsc.load_gather(
                    pad_v, [iota_p + (bc * (LANES * PITCH) + d)])
                acc_v[pl.ds(off_d + bc * LANES, LANES)] = v

    def fire_out(acc_v, l, sem):
        for td in range(TD):
            pltpu.async_copy(
                acc_v.at[pl.ds(td * 1024, 1024)],
                out_hbm.at[pl.ds((l + 1) * LSTRIDE + td * NW * 1024
                                 + wid * 1024, 1024)], sem)

    fire_gather(0, bin_a, sub_a, sem_a)
    fire_gather(1, bin_b, sub_b, sem_b)

    def pair_body(g, carry):
        l0 = 2 * g
        drain_gather(bin_a, sub_a, sem_a)

        @pl.when(g > 0)
        def _():
            drain_out(acc_a, sem_oa)

        stage_sum(bin_a, sub_a, l0)

        @pl.when(g < L // 2 - 1)
        def _():
            fire_gather(l0 + 2, bin_a, sub_a, sem_a)

        stage_tiles(acc_a)
        fire_out(acc_a, l0, sem_oa)

        drain_gather(bin_b, sub_b, sem_b)

        @pl.when(g > 0)
        def _():
            drain_out(acc_b, sem_ob)

        stage_sum(bin_b, sub_b, l0 + 1)

        @pl.when(g < L // 2 - 1)
        def _():
            fire_gather(l0 + 3, bin_b, sub_b, sem_b)

        stage_tiles(acc_b)
        fire_out(acc_b, l0 + 1, sem_ob)
        return carry

    lax.fori_loop(0, L // 2, pair_body, 0)
    drain_out(acc_a, sem_oa)
    drain_out(acc_b, sem_ob)


def kernel(bin_ids, subbin_ids, pos_table, bin_table, subbin_table, cls_table):
    def phys_ids(ids):
        return (ids.astype(jnp.int32).T.reshape(LT, 8, NW, 128)
                .transpose(0, 2, 1, 3).reshape(-1))

    flat = _emb_kernel(phys_ids(bin_ids), phys_ids(subbin_ids),
                       pos_table.reshape(-1), bin_table, subbin_table,
                       cls_table.reshape(-1))
    out5 = flat.reshape(L + 1, TD, NW, 8, 128)
    return out5.transpose(2, 4, 0, 1, 3).reshape(B, L + 1, D)
